# Initial kernel scaffold; baseline (speedup 1.0000x reference)
import jax
import jax.numpy as jnp
from jax.experimental import pallas as pl


def kernel(x, W_fc, b_fc, U0, U1, U2, S0, S1, S2, Wc0, bc0, Wc1, bc1, Wc2, bc2):
    raise NotImplementedError("write your pallas kernel here")


# speedup vs baseline: 2.3311x; 2.3311x over previous
import functools
import math

import jax
import jax.numpy as jnp
from jax import lax
from jax.experimental import pallas as pl
from jax.experimental.pallas import tpu as pltpu
from jax.experimental.pallas import tpu_sc as plsc

_NC, _NS = 2, 16
_NW = _NC * _NS


def _cdiv(a, b):
  return (a + b - 1) // b


def _mm(x, w, *, bias=None, act=None, mask=None, mb=None):
  M, K = x.shape
  N = w.shape[1]
  if mb is None or mb >= M:
    mb = M
  grid = (_cdiv(M, mb),)
  in_specs = [
      pl.BlockSpec((mb, K), lambda i: (i, 0)),
      pl.BlockSpec((K, N), lambda i: (0, 0)),
  ]
  args = [x, w]
  if bias is not None:
    in_specs.append(pl.BlockSpec((1, N), lambda i: (0, 0)))
    args.append(bias.reshape(1, N))
  if mask is not None:
    in_specs.append(pl.BlockSpec((mb, 1), lambda i: (i, 0)))
    args.append(mask)

  def kern(*refs):
    x_ref, w_ref = refs[0], refs[1]
    o_ref = refs[-1]
    acc = jnp.dot(x_ref[...], w_ref[...], preferred_element_type=jnp.float32)
    p = 2
    if bias is not None:
      acc = acc + refs[p][...]
      p += 1
    if act == 'elu':
      acc = jnp.where(acc > 0, acc, jnp.exp(jnp.minimum(acc, 0.0)) - 1.0)
    if mask is not None:
      acc = acc * refs[p][...]
      p += 1
    o_ref[...] = acc

  return pl.pallas_call(
      kern,
      grid=grid,
      in_specs=in_specs,
      out_specs=pl.BlockSpec((mb, N), lambda i: (i, 0)),
      out_shape=jax.ShapeDtypeStruct((M, N), jnp.float32),
  )(*args)


def _sc_gather(table, gidx, F, C):
  Gp = gidx.shape[0]
  nch = Gp // C
  rounds = _cdiv(nch, _NW)
  mesh = plsc.VectorSubcoreMesh(
      core_axis_name="c", subcore_axis_name="s",
      num_cores=_NC, num_subcores=_NS)

  @functools.partial(
      pl.kernel,
      out_type=jax.ShapeDtypeStruct((Gp, F), jnp.float32),
      mesh=mesh,
      scratch_types=[
          pltpu.VMEM((C,), jnp.int32),
          pltpu.VMEM((C, F), jnp.float32),
          pltpu.SemaphoreType.DMA,
      ],
  )
  def gk(table_hbm, gidx_hbm, out_hbm, idx_v, rows_v, sem):
    wid = lax.axis_index("s") * _NC + lax.axis_index("c")

    def body(j, carry):
      ch = j * _NW + wid

      @pl.when(ch < nch)
      def _():
        base = ch * C
        pltpu.sync_copy(gidx_hbm.at[pl.ds(base, C)], idx_v)
        pltpu.async_copy(table_hbm.at[idx_v], rows_v, sem).wait()
        pltpu.sync_copy(rows_v, out_hbm.at[pl.ds(base, C)])

      return carry

    lax.fori_loop(0, rounds, body, 0)

  return gk(table, gidx)


_LEVEL_TUNE = [
    (128, None, None),
    (256, None, 640),
    (512, 1112, 1264),
]


def kernel(x, W_fc, b_fc, U0, U1, U2, S0, S1, S2, Wc0, bc0, Wc1, bc1, Wc2, bc2):
  B = x.shape[0]
  h = _mm(x, W_fc, bias=b_fc)
  M_in = U2.shape[1]
  F_in = h.shape[1] // M_in
  hcol = h.reshape(B, M_in, F_in).transpose(1, 0, 2).reshape(M_in, B * F_in)

  specs = [
      (U2, S2, Wc0, bc0, 'elu'),
      (U1, S1, Wc1, bc1, 'elu'),
      (U0, S0, Wc2, bc2, None),
  ]
  out_col = None
  for (U, S, Wc, bc, act), (C, mb_up, mb_cv) in zip(specs, _LEVEL_TUNE):
    M = U.shape[0]
    F = hcol.shape[1] // B
    sl = S.shape[-1]
    OC = Wc.shape[1]

    y = _mm(U, hcol, mb=mb_up)

    gidx = S[0].reshape(-1)
    G = gidx.shape[0]
    step = math.lcm(C, sl)
    Gp = _cdiv(G, step) * step
    if Gp != G:
      gidx = jnp.concatenate([gidx, jnp.zeros((Gp - G,), jnp.int32)])
    gout = _sc_gather(y, gidx, B * F, C)
    R = Gp // sl
    gmat = gout.reshape(R, sl * B * F)

    W3 = Wc.reshape(sl, F, OC)
    eyeB = jnp.eye(B, dtype=jnp.float32)
    W2 = jnp.einsum('sco,bd->sbcdo', W3, eyeB).reshape(sl * B * F, B * OC)
    b2 = jnp.tile(bc, B)

    rows = jnp.arange(R, dtype=jnp.int32)
    mcol = jnp.where(rows == M - 1, 0.0, 1.0).astype(jnp.float32).reshape(R, 1)

    out_col = _mm(gmat, W2, bias=b2, act=act, mask=mcol, mb=mb_cv)
    hcol = out_col[:M]
    M_last, OC_last = M, OC
  return (out_col[:M_last]
          .reshape(M_last, B, OC_last)
          .transpose(1, 0, 2))
